# Initial kernel scaffold; baseline (speedup 1.0000x reference)
import jax
import jax.numpy as jnp
from jax.experimental import pallas as pl


def kernel(q_ids, s_ids, table, W1, b1, W2, b2, W3, b3):
    raise NotImplementedError("write your pallas kernel here")


# speedup vs baseline: 2.2105x; 2.2105x over previous
import functools

import jax
import jax.numpy as jnp
from jax import lax
from jax.experimental import pallas as pl
from jax.experimental.pallas import tpu as pltpu
from jax.experimental.pallas import tpu_sc as plsc

VOCAB = 1000000
D = 64
B = 16384
LQ = 20
LS = 50

_info = plsc.get_sparse_core_info()
NC = _info.num_cores
NS = _info.num_subcores
NW = NC * NS
BAGS_PER_W = B // NW
CB = 2
NCHUNK = BAGS_PER_W // CB


def _embed_bags(q_ids3, s_ids3, table):
    mesh = plsc.VectorSubcoreMesh(core_axis_name="c", subcore_axis_name="s")

    @functools.partial(
        pl.kernel,
        mesh=mesh,
        out_type=jax.ShapeDtypeStruct((B, 2 * D), jnp.float32),
        scratch_types=[
            pltpu.VMEM((NCHUNK, CB * LQ), jnp.int32),
            pltpu.VMEM((NCHUNK, CB * LS), jnp.int32),
            pltpu.VMEM((CB * LQ, D), jnp.float32),
            pltpu.VMEM((CB * LS, D), jnp.float32),
            pltpu.VMEM((BAGS_PER_W, 2 * D), jnp.float32),
            pltpu.SemaphoreType.DMA,
            pltpu.SemaphoreType.DMA,
        ],
        compiler_params=pltpu.CompilerParams(use_tc_tiling_on_sc=False),
    )
    def sc_kernel(q_hbm, s_hbm, table_hbm, out_hbm,
                  qidx_v, sidx_v, qrows_v, srows_v, h_v, qsem, ssem):
        wid = lax.axis_index("s") * NC + lax.axis_index("c")
        pltpu.sync_copy(q_hbm.at[wid], qidx_v)
        pltpu.sync_copy(s_hbm.at[wid], sidx_v)

        def chunk_body(j, _):
            cq = pltpu.async_copy(table_hbm.at[qidx_v.at[j]], qrows_v, qsem)
            cs = pltpu.async_copy(table_hbm.at[sidx_v.at[j]], srows_v, ssem)
            cq.wait()
            cs.wait()
            for bag in range(CB):
                for c in range(D // 16):
                    sl = pl.ds(c * 16, 16)
                    acc = qrows_v[bag * LQ, sl]
                    for r in range(1, LQ):
                        acc = acc + qrows_v[bag * LQ + r, sl]
                    h_v[j * CB + bag, sl] = acc * (1.0 / LQ)
                    acc = srows_v[bag * LS, sl]
                    for r in range(1, LS):
                        acc = acc + srows_v[bag * LS + r, sl]
                    h_v[j * CB + bag, pl.ds(D + c * 16, 16)] = acc * (1.0 / LS)
            return 0

        lax.fori_loop(0, NCHUNK, chunk_body, 0)
        pltpu.sync_copy(h_v, out_hbm.at[pl.ds(wid * BAGS_PER_W, BAGS_PER_W)])

    return sc_kernel(q_ids3, s_ids3, table)


def _mlp_body(h_ref, W1_ref, b1_ref, W2_ref, b2_ref, W3_ref, b3_ref, out_ref):
    h = h_ref[...]
    z1 = jnp.maximum(
        jnp.dot(h, W1_ref[...].T, preferred_element_type=jnp.float32) + b1_ref[...], 0.0)
    z2 = jnp.maximum(
        jnp.dot(z1, W2_ref[...].T, preferred_element_type=jnp.float32) + b2_ref[...], 0.0)
    out_ref[...] = jnp.sum(z2 * W3_ref[...], axis=1, keepdims=True) + b3_ref[0]


def _mlp(h, W1, b1, W2, b2, W3, b3):
    BLK = 1024
    grid = (B // BLK,)
    return pl.pallas_call(
        _mlp_body,
        grid=grid,
        in_specs=[
            pl.BlockSpec((BLK, 2 * D), lambda i: (i, 0)),
            pl.BlockSpec((128, 2 * D), lambda i: (0, 0)),
            pl.BlockSpec((128,), lambda i: (0,)),
            pl.BlockSpec((32, 128), lambda i: (0, 0)),
            pl.BlockSpec((32,), lambda i: (0,)),
            pl.BlockSpec((1, 32), lambda i: (0, 0)),
            pl.BlockSpec((1,), lambda i: (0,)),
        ],
        out_specs=pl.BlockSpec((BLK, 1), lambda i: (i, 0)),
        out_shape=jax.ShapeDtypeStruct((B, 1), jnp.float32),
        compiler_params=pltpu.CompilerParams(
            dimension_semantics=("parallel",),
        ),
    )(h, W1, b1, W2, b2, W3, b3)


def kernel(q_ids, s_ids, table, W1, b1, W2, b2, W3, b3):
    q3 = q_ids.reshape(NW, NCHUNK, CB * LQ).astype(jnp.int32)
    s3 = s_ids.reshape(NW, NCHUNK, CB * LS).astype(jnp.int32)
    h = _embed_bags(q3, s3, table)
    out = _mlp(h, W1, b1, W2, b2, W3, b3)
    return out.squeeze(-1)
